# Initial kernel scaffold; baseline (speedup 1.0000x reference)
import jax
import jax.numpy as jnp
from jax.experimental import pallas as pl


def kernel(x_note, edge_index_onset, edge_index_consecutive, edge_attr_onset, edge_attr_consecutive, params):
    raise NotImplementedError("write your pallas kernel here")


# speedup vs baseline: 2.5170x; 2.5170x over previous
import functools
import jax
import jax.numpy as jnp
from jax import lax
from jax.experimental import pallas as pl
from jax.experimental.pallas import tpu as pltpu
from jax.experimental.pallas import tpu_sc as plsc

N_NODES = 10000
N_PAD = 10240
D = 128
D_EDGE = 16
E = 320000

NC = 2
NS = 16
NW = NC * NS
CH = 80
E_PER_TILE = E // NW
NCHUNK = E_PER_TILE // CH
STRIPE = N_PAD // NS
ZCH = 32
DCH = 32

_HIGH = jax.lax.Precision.HIGHEST


def _dot(a, b):
    return jnp.dot(a, b, precision=_HIGH, preferred_element_type=jnp.float32)


def _edge_mlp_body(a_ref, w_ref, b_ref, o_ref):
    o_ref[...] = jax.nn.relu(_dot(a_ref[...], w_ref[...]) + b_ref[...])


def _edge_mlp(a, w, b, d_in, n_blk):
    return pl.pallas_call(
        _edge_mlp_body,
        grid=(E // n_blk,),
        in_specs=[
            pl.BlockSpec((n_blk, d_in), lambda i: (i, 0)),
            pl.BlockSpec((d_in, D), lambda i: (0, 0)),
            pl.BlockSpec((1, D), lambda i: (0, 0)),
        ],
        out_specs=pl.BlockSpec((n_blk, D), lambda i: (i, 0)),
        out_shape=jax.ShapeDtypeStruct((E, D), jnp.float32),
    )(a, w, b)


def _combine_body(xs_on, g_on, xs_co, g_co, deg_on, deg_co,
                  m1_ref, m2_ref, c_ref, bias_ref, o_ref, *, activate):
    acc = bias_ref[0]
    acc = acc + _dot(xs_on[0] + xs_on[1], m1_ref[0])
    acc = acc + _dot(g_on[0] + g_on[1], m2_ref[0])
    acc = acc + _dot(xs_co[0] + xs_co[1], m1_ref[1])
    acc = acc + _dot(g_co[0] + g_co[1], m2_ref[1])
    acc = acc + (deg_on[0, :, 0:1] + deg_on[1, :, 0:1]) * c_ref[0]
    acc = acc + (deg_co[0, :, 0:1] + deg_co[1, :, 0:1]) * c_ref[1]
    if activate:
        acc = jax.nn.relu(acc)
        nrm = jnp.sqrt(jnp.sum(acc * acc, axis=-1, keepdims=True))
        acc = acc / jnp.maximum(nrm, 1e-12)
    o_ref[...] = acc


def _combine(xs_on, g_on, xs_co, g_co, deg_on, deg_co, m1, m2, c, bias,
             activate):
    n_blk = 400
    nd = pl.BlockSpec((2, n_blk, D), lambda i: (0, i, 0))
    dg = pl.BlockSpec((2, n_blk, D), lambda i: (0, i, 0))
    return pl.pallas_call(
        functools.partial(_combine_body, activate=activate),
        grid=(N_NODES // n_blk,),
        in_specs=[
            nd, nd, nd, nd, dg, dg,
            pl.BlockSpec((2, D, D), lambda i: (0, 0, 0)),
            pl.BlockSpec((2, D, D), lambda i: (0, 0, 0)),
            pl.BlockSpec((2, 1, D), lambda i: (0, 0, 0)),
            pl.BlockSpec((1, 1, D), lambda i: (0, 0, 0)),
        ],
        out_specs=pl.BlockSpec((n_blk, D), lambda i: (i, 0)),
        out_shape=jax.ShapeDtypeStruct((N_NODES, D), jnp.float32),
    )(xs_on, g_on, xs_co, g_co, deg_on, deg_co, m1, m2, c, bias)


def _zero16():
    return jnp.zeros((16,), jnp.float32)


def _fill_zb(zb):
    def body(j, _):
        for k in range(D // 16):
            zb[j, pl.ds(k * 16, 16)] = _zero16()
        return 0
    lax.fori_loop(0, ZCH, body, 0)


def _zero_acc(acc, zb, s):
    def body(m, _):
        pltpu.sync_copy(zb, acc.at[pl.ds(s * STRIPE + m * ZCH, ZCH)])
        return 0
    lax.fori_loop(0, STRIPE // ZCH, body, 0)


def _dump_acc(acc, out_hbm, c, s, bounce):
    def body(m, _):
        row = s * STRIPE + m * ZCH
        pltpu.sync_copy(acc.at[pl.ds(row, ZCH)], bounce)
        pltpu.sync_copy(bounce, out_hbm.at[c, pl.ds(row, ZCH)])
        return 0
    lax.fori_loop(0, STRIPE // ZCH, body, 0)


def _sc_mesh():
    return plsc.VectorSubcoreMesh(core_axis_name="c", subcore_axis_name="s",
                                  num_cores=NC, num_subcores=NS)


def _sc_deg(dst):
    out_type = jax.ShapeDtypeStruct((2, N_PAD, D), jnp.float32)
    scratch = [
        pltpu.VMEM((CH,), jnp.int32),
        pltpu.VMEM((CH, D), jnp.float32),
        pltpu.VMEM((ZCH, D), jnp.float32),
        pltpu.VMEM_SHARED((N_PAD, D), jnp.float32),
    ]

    @functools.partial(pl.kernel, out_type=out_type, mesh=_sc_mesh(),
                       scratch_types=scratch)
    def k(dst_h, deg_out, didx, ones_v, zb, acc):
        c = lax.axis_index("c")
        s = lax.axis_index("s")
        w = c * NS + s

        _fill_zb(zb)

        def body_ones(j, _):
            for kk in range(D // 16):
                ones_v[j, pl.ds(kk * 16, 16)] = jnp.ones((16,), jnp.float32)
            return 0
        lax.fori_loop(0, CH, body_ones, 0)

        _zero_acc(acc, zb, s)
        plsc.subcore_barrier()

        def body(i, _):
            base = w * E_PER_TILE + i * CH
            pltpu.sync_copy(dst_h.at[pl.ds(base, CH)], didx)
            pltpu.sync_copy(ones_v, acc.at[didx], add=True)
            return 0
        lax.fori_loop(0, NCHUNK, body, 0)
        plsc.subcore_barrier()
        _dump_acc(acc, deg_out, c, s, zb)

    return k(dst)


def _sc_segsum(rows1, dst):
    out_type = jax.ShapeDtypeStruct((2, N_PAD, D), jnp.float32)
    scratch = [
        pltpu.VMEM((CH,), jnp.int32),
        pltpu.VMEM((CH, D), jnp.float32),
        pltpu.VMEM((ZCH, D), jnp.float32),
        pltpu.VMEM_SHARED((N_PAD, D), jnp.float32),
    ]

    @functools.partial(pl.kernel, out_type=out_type, mesh=_sc_mesh(),
                       scratch_types=scratch)
    def k(rows_h, dst_h, g_out, didx, rows, zb, acc):
        c = lax.axis_index("c")
        s = lax.axis_index("s")
        w = c * NS + s

        _fill_zb(zb)
        _zero_acc(acc, zb, s)
        plsc.subcore_barrier()

        def body(i, _):
            base = w * E_PER_TILE + i * CH
            pltpu.sync_copy(dst_h.at[pl.ds(base, CH)], didx)
            pltpu.sync_copy(rows_h.at[pl.ds(base, CH)], rows)
            pltpu.sync_copy(rows, acc.at[didx], add=True)
            return 0
        lax.fori_loop(0, NCHUNK, body, 0)
        plsc.subcore_barrier()
        _dump_acc(acc, g_out, c, s, zb)

    return k(rows1, dst)


def _sc_gather_segsum(table, src, dst):
    out_type = jax.ShapeDtypeStruct((2, N_PAD, D), jnp.float32)
    scratch = [
        pltpu.VMEM((CH,), jnp.int32),
        pltpu.VMEM((CH,), jnp.int32),
        pltpu.VMEM((CH, D), jnp.float32),
        pltpu.VMEM((ZCH, D), jnp.float32),
        pltpu.VMEM_SHARED((N_PAD, D), jnp.float32),
        pltpu.SemaphoreType.DMA,
    ]

    @functools.partial(pl.kernel, out_type=out_type, mesh=_sc_mesh(),
                       scratch_types=scratch)
    def k(table_h, src_h, dst_h, xs_out, sidx, didx, rows, zb, acc, sem):
        c = lax.axis_index("c")
        s = lax.axis_index("s")
        w = c * NS + s

        _fill_zb(zb)
        _zero_acc(acc, zb, s)
        plsc.subcore_barrier()

        def body(i, _):
            base = w * E_PER_TILE + i * CH
            pltpu.sync_copy(src_h.at[pl.ds(base, CH)], sidx)
            pltpu.sync_copy(dst_h.at[pl.ds(base, CH)], didx)
            pltpu.async_copy(table_h.at[sidx], rows, sem).wait()
            pltpu.sync_copy(rows, acc.at[didx], add=True)
            return 0
        lax.fori_loop(0, NCHUNK, body, 0)
        plsc.subcore_barrier()
        _dump_acc(acc, xs_out, c, s, zb)

    return k(table, src, dst)


def _sc_layer1_gather(h, src, dst):
    out_type = (
        jax.ShapeDtypeStruct((E, D), jnp.float32),
        jax.ShapeDtypeStruct((2, N_PAD, D), jnp.float32),
    )
    scratch = [
        pltpu.VMEM((CH,), jnp.int32),
        pltpu.VMEM((CH,), jnp.int32),
        pltpu.VMEM((CH, D), jnp.float32),
        pltpu.VMEM((CH, D), jnp.float32),
        pltpu.VMEM((ZCH, D), jnp.float32),
        pltpu.VMEM_SHARED((N_PAD, D), jnp.float32),
        pltpu.SemaphoreType.DMA,
        pltpu.SemaphoreType.DMA,
    ]

    @functools.partial(pl.kernel, out_type=out_type, mesh=_sc_mesh(),
                       scratch_types=scratch)
    def k(h_h, src_h, dst_h, dm_out, xs_out, sidx, didx, rs, rd, zb, acc,
          sem1, sem2):
        c = lax.axis_index("c")
        s = lax.axis_index("s")
        w = c * NS + s

        _fill_zb(zb)
        _zero_acc(acc, zb, s)
        plsc.subcore_barrier()

        def body(i, _):
            base = w * E_PER_TILE + i * CH
            pltpu.sync_copy(src_h.at[pl.ds(base, CH)], sidx)
            pltpu.sync_copy(dst_h.at[pl.ds(base, CH)], didx)
            cp1 = pltpu.async_copy(h_h.at[sidx], rs, sem1)
            cp2 = pltpu.async_copy(h_h.at[didx], rd, sem2)
            cp1.wait()
            cp2.wait()
            pltpu.sync_copy(rs, acc.at[didx], add=True)

            def body_row(j, _):
                for kk in range(D // 16):
                    sl = pl.ds(kk * 16, 16)
                    rs[j, sl] = jnp.abs(rs[j, sl] - rd[j, sl])
                return 0
            lax.fori_loop(0, CH, body_row, 0)
            pltpu.sync_copy(rs, dm_out.at[pl.ds(base, CH)])
            return 0
        lax.fori_loop(0, NCHUNK, body, 0)
        plsc.subcore_barrier()
        _dump_acc(acc, xs_out, c, s, zb)

    return k(h, src, dst)


def _fold(p):
    wp = p["proj"]["w"]
    wp1, wp2 = wp[:, :D], wp[:, D:]
    m1 = p["lin"]["w"].T @ wp1.T
    m2 = p["e2"]["w"].T @ wp2.T
    c = p["proj"]["b"] + p["lin"]["b"] @ wp1.T + p["e2"]["b"] @ wp2.T
    return m1, m2, c


def _layer_consts(lp):
    folds = [_fold(lp[r]) for r in ("onset", "consecutive")]
    m1 = jnp.stack([f[0] for f in folds])
    m2 = jnp.stack([f[1] for f in folds])
    c = jnp.stack([f[2] for f in folds])[:, None, :]
    bias = (lp["onset"]["bias"] + lp["consecutive"]["bias"])[None, None, :]
    w1 = [lp[r]["e1"]["w"].T for r in ("onset", "consecutive")]
    b1 = [lp[r]["e1"]["b"][None, :] for r in ("onset", "consecutive")]
    return m1, m2, c, bias, w1, b1


def kernel(x_note, edge_index_onset, edge_index_consecutive,
           edge_attr_onset, edge_attr_consecutive, params):
    src_on, dst_on = edge_index_onset[0], edge_index_onset[1]
    src_co, dst_co = edge_index_consecutive[0], edge_index_consecutive[1]

    m1_0, m2_0, c_0, bias_0, w1_0, b1_0 = _layer_consts(params["l0"])
    m1_1, m2_1, c_1, bias_1, w1_1, b1_1 = _layer_consts(params["l1"])

    e0p_on = _edge_mlp(edge_attr_onset, w1_0[0], b1_0[0], D_EDGE, 2560)
    e0p_co = _edge_mlp(edge_attr_consecutive, w1_0[1], b1_0[1], D_EDGE, 2560)
    g0_on = _sc_segsum(e0p_on, dst_on)
    g0_co = _sc_segsum(e0p_co, dst_co)
    deg_on = _sc_deg(dst_on)
    deg_co = _sc_deg(dst_co)
    xs0_on = _sc_gather_segsum(x_note, src_on, dst_on)
    xs0_co = _sc_gather_segsum(x_note, src_co, dst_co)
    h = _combine(xs0_on, g0_on, xs0_co, g0_co, deg_on, deg_co,
                 m1_0, m2_0, c_0, bias_0, activate=True)

    dm_on, xs1_on = _sc_layer1_gather(h, src_on, dst_on)
    dm_co, xs1_co = _sc_layer1_gather(h, src_co, dst_co)
    e1p_on = _edge_mlp(dm_on, w1_1[0], b1_1[0], D, 1280)
    e1p_co = _edge_mlp(dm_co, w1_1[1], b1_1[1], D, 1280)
    g1_on = _sc_segsum(e1p_on, dst_on)
    g1_co = _sc_segsum(e1p_co, dst_co)
    out = _combine(xs1_on, g1_on, xs1_co, g1_co, deg_on, deg_co,
                   m1_1, m2_1, c_1, bias_1, activate=False)
    return out
